# Initial kernel scaffold; baseline (speedup 1.0000x reference)
import jax
import jax.numpy as jnp
from jax.experimental import pallas as pl


def kernel(inputs, edge_index, edge_attr, W1, b1, W2, b2):
    raise NotImplementedError("write your pallas kernel here")


# speedup vs baseline: 3.8754x; 3.8754x over previous
import functools

import jax
import jax.numpy as jnp
from jax import lax
from jax.experimental import pallas as pl
from jax.experimental.pallas import tpu as pltpu
from jax.experimental.pallas import tpu_sc as plsc

N_NODES = 10000
N_PAD = 10240
N_EDGES = 320000
D_IN = 128
D_HID = 128
D_OUT = 64

NC = 2
NS = 16
LANES = 16
NW = NC * NS

EDGE_CHUNK = 80


def _sc_aggregate_body(n_nodes, d, chunks_per_worker,
                       h_hbm, src_hbm, dst_hbm, w_hbm, zeros_hbm, out_hbm,
                       src_v, dst_v, w_v, rows_v, acc, sem):
    c = lax.axis_index("c")
    s = lax.axis_index("s")
    wid = c * NS + s
    rows_per_tile = n_nodes // NS

    pltpu.sync_copy(zeros_hbm.at[pl.ds(s * rows_per_tile, rows_per_tile)],
                    acc.at[pl.ds(s * rows_per_tile, rows_per_tile)])
    plsc.subcore_barrier()

    edge0 = wid * (chunks_per_worker * EDGE_CHUNK)

    def chunk_body(k, carry):
        off = edge0 + k * EDGE_CHUNK
        pltpu.sync_copy(src_hbm.at[pl.ds(off, EDGE_CHUNK)], src_v)
        pltpu.sync_copy(dst_hbm.at[pl.ds(off, EDGE_CHUNK)], dst_v)
        pltpu.sync_copy(w_hbm.at[pl.ds(off, EDGE_CHUNK)], w_v)
        pltpu.async_copy(h_hbm.at[src_v], rows_v, sem).wait()

        def group_body(g, carry2):
            wvec = w_v[pl.ds(g * LANES, LANES)]
            for i in range(LANES):
                wsplat = jnp.full((LANES,), wvec[i], dtype=jnp.float32)
                r = g * LANES + i
                for j in range(d // LANES):
                    sl = pl.ds(j * LANES, LANES)
                    rows_v[r, sl] = rows_v[r, sl] * wsplat
            return carry2

        lax.fori_loop(0, EDGE_CHUNK // LANES, group_body, None)

        pltpu.sync_copy(rows_v, acc.at[dst_v], add=True)
        return carry

    lax.fori_loop(0, chunks_per_worker, chunk_body, None)
    plsc.subcore_barrier()

    pltpu.sync_copy(acc.at[pl.ds(s * rows_per_tile, rows_per_tile)],
                    out_hbm.at[c, pl.ds(s * rows_per_tile, rows_per_tile)])


def _sc_aggregate(h, src2d, dst2d, w2d, zeros):
    n_nodes, d = h.shape
    chunks_per_worker = src2d.shape[0] // (NW * EDGE_CHUNK)
    mesh = plsc.VectorSubcoreMesh(core_axis_name="c", subcore_axis_name="s",
                                  num_cores=NC, num_subcores=NS)
    body = functools.partial(_sc_aggregate_body, n_nodes, d, chunks_per_worker)
    return pl.kernel(
        body,
        out_type=jax.ShapeDtypeStruct((NC, n_nodes, d), jnp.float32),
        mesh=mesh,
        scratch_types=[
            pltpu.VMEM((EDGE_CHUNK,), jnp.int32),
            pltpu.VMEM((EDGE_CHUNK,), jnp.int32),
            pltpu.VMEM((EDGE_CHUNK,), jnp.float32),
            pltpu.VMEM((EDGE_CHUNK, d), jnp.float32),
            pltpu.VMEM_SHARED((n_nodes, d), jnp.float32),
            pltpu.SemaphoreType.DMA,
        ],
        compiler_params=pltpu.CompilerParams(use_tc_tiling_on_sc=False),
    )(h, src2d, dst2d, w2d, zeros)


_ROW_BLOCK = 1024


def _mm_body(x_ref, w_ref, o_ref):
    o_ref[...] = jnp.dot(x_ref[...], w_ref[...],
                         preferred_element_type=jnp.float32)


def _tc_matmul(x, w):
    m, k = x.shape
    n = w.shape[1]
    grid = (m // _ROW_BLOCK,)
    return pl.pallas_call(
        _mm_body,
        grid=grid,
        in_specs=[
            pl.BlockSpec((_ROW_BLOCK, k), lambda i: (i, 0)),
            pl.BlockSpec((k, n), lambda i: (0, 0)),
        ],
        out_specs=pl.BlockSpec((_ROW_BLOCK, n), lambda i: (i, 0)),
        out_shape=jax.ShapeDtypeStruct((m, n), jnp.float32),
    )(x, w)


def _relu_mm_body(p_ref, b_ref, w_ref, o_ref):
    h = jnp.maximum(p_ref[0] + p_ref[1] + b_ref[...], 0.0)
    o_ref[...] = jnp.dot(h, w_ref[...], preferred_element_type=jnp.float32)


def _tc_relu_matmul(partials, b, w):
    _, m, k = partials.shape
    n = w.shape[1]
    grid = (m // _ROW_BLOCK,)
    return pl.pallas_call(
        _relu_mm_body,
        grid=grid,
        in_specs=[
            pl.BlockSpec((NC, _ROW_BLOCK, k), lambda i: (0, i, 0)),
            pl.BlockSpec((1, k), lambda i: (0, 0)),
            pl.BlockSpec((k, n), lambda i: (0, 0)),
        ],
        out_specs=pl.BlockSpec((_ROW_BLOCK, n), lambda i: (i, 0)),
        out_shape=jax.ShapeDtypeStruct((m, n), jnp.float32),
    )(partials, b.reshape(1, k), w)


def _logsoftmax_body(p_ref, b_ref, o_ref):
    z = p_ref[0] + p_ref[1] + b_ref[...]
    m = jnp.max(z, axis=1, keepdims=True)
    e = jnp.exp(z - m)
    ssum = jnp.sum(e, axis=1, keepdims=True)
    o_ref[...] = z - m - jnp.log(ssum)


def _tc_logsoftmax(partials, b):
    _, m, n = partials.shape
    grid = (m // _ROW_BLOCK,)
    return pl.pallas_call(
        _logsoftmax_body,
        grid=grid,
        in_specs=[
            pl.BlockSpec((NC, _ROW_BLOCK, n), lambda i: (0, i, 0)),
            pl.BlockSpec((1, n), lambda i: (0, 0)),
        ],
        out_specs=pl.BlockSpec((_ROW_BLOCK, n), lambda i: (i, 0)),
        out_shape=jax.ShapeDtypeStruct((m, n), jnp.float32),
    )(partials, b.reshape(1, n))


def kernel(inputs, edge_index, edge_attr, W1, b1, W2, b2):
    src = edge_index[0].astype(jnp.int32)
    dst = edge_index[1].astype(jnp.int32)
    w2d = edge_attr
    zeros_hid = jnp.zeros((N_PAD, D_HID), jnp.float32)
    zeros_out = jnp.zeros((N_PAD, D_OUT), jnp.float32)
    x_p = jnp.pad(inputs, ((0, N_PAD - N_NODES), (0, 0)))

    h = _tc_matmul(x_p, W1)
    p1 = _sc_aggregate(h, src, dst, w2d, zeros_hid)
    h2 = _tc_relu_matmul(p1, b1, W2)
    p2 = _sc_aggregate(h2, src, dst, w2d, zeros_out)
    return _tc_logsoftmax(p2, b2)[:N_NODES]
